# baseline (speedup 1.0000x reference)
import functools

import jax
import jax.numpy as jnp
from jax import lax
from jax.experimental import pallas as pl
from jax.experimental.pallas import tpu as pltpu
from jax.experimental.pallas import tpu_sc as plsc

N = 10000
D = 128
E = 320000
NC = 2
NS = 16
NW = NC * NS

CH = 64
SPB = 2
NBLK = 80
NP = NBLK // 2
EPW = NBLK * SPB * CH
EPW_DEG = E // NW
PAD_PW = EPW - EPW_DEG
ROWS_PT = 624
ROW_TAIL = N - NS * ROWS_PT
N_ACC = 10008
N_HIST = 10240

@functools.cache
def _sc_kernels():
    mesh = plsc.VectorSubcoreMesh(
        core_axis_name="c", subcore_axis_name="s", num_cores=NC, num_subcores=NS
    )
    sc_params = pltpu.CompilerParams(needs_layout_passes=False)
    deg_kernel = functools.partial(
        pl.kernel,
        out_type=jax.ShapeDtypeStruct((NW, N_HIST), jnp.float32),
        mesh=mesh,
        compiler_params=sc_params,
        scratch_types=[
            pltpu.VMEM((N_HIST,), jnp.float32),
            pltpu.VMEM((EPW_DEG,), jnp.int32),
        ],
    )(_deg_body)
    seg_kernel = functools.partial(
        pl.kernel,
        out_type=jax.ShapeDtypeStruct((NC, N, D), jnp.float32),
        mesh=mesh,
        compiler_params=sc_params,
        scratch_types=[pltpu.VMEM_SHARED((N_ACC, D), jnp.float32)]
        + [pltpu.VMEM((SPB, CH), jnp.int32) for _ in range(4)]
        + [pltpu.VMEM((CH, D), jnp.float32) for _ in range(2 * SPB)]
        + [pltpu.SemaphoreType.DMA for _ in range(4 * SPB + 4)],
    )(_seg_body)
    return deg_kernel, seg_kernel


def _deg_body(dst_hbm, out_hbm, hist, dstv):
    cid = lax.axis_index("c")
    sid = lax.axis_index("s")
    wid = sid * NC + cid

    zeros16 = jnp.zeros((16,), jnp.float32)

    def zbody(i, c):
        hist[pl.ds(i * 16, 16)] = zeros16
        return c

    lax.fori_loop(0, N_HIST // 16, zbody, 0)

    off = pl.multiple_of(wid * EPW_DEG, 8)
    pltpu.sync_copy(dst_hbm.at[pl.ds(off, EPW_DEG)], dstv)

    ones16 = jnp.ones((16,), jnp.float32)

    def body(i, c):
        idx = dstv[pl.ds(i * 16, 16)]
        plsc.addupdate_scatter(hist, [idx], ones16)
        return c

    lax.fori_loop(0, EPW_DEG // 16, body, 0)
    pltpu.sync_copy(hist, out_hbm.at[wid])


def _seg_body(xs_hbm, srcp_hbm, dstp_hbm, out_hbm, acc, *rest):
    sidx = rest[0:2]
    didx = rest[2:4]
    rows = [rest[4 : 4 + SPB], rest[4 + SPB : 4 + 2 * SPB]]
    p = 4 + 2 * SPB
    gsem = [rest[p : p + SPB], rest[p + SPB : p + 2 * SPB]]
    p += 2 * SPB
    ssem = [rest[p : p + SPB], rest[p + SPB : p + 2 * SPB]]
    p += 2 * SPB
    is_sem = rest[p : p + 2]
    id_sem = rest[p + 2 : p + 4]

    cid = lax.axis_index("c")
    sid = lax.axis_index("s")
    wid = sid * NC + cid

    r0 = pl.multiple_of(sid * ROWS_PT, 8)
    pltpu.sync_copy(xs_hbm.at[pl.ds(r0, ROWS_PT)], acc.at[pl.ds(r0, ROWS_PT)])

    @pl.when(sid == 0)
    def _():
        t0 = NS * ROWS_PT
        pltpu.sync_copy(
            xs_hbm.at[pl.ds(t0, ROW_TAIL)], acc.at[pl.ds(t0, ROW_TAIL)]
        )

    pltpu.sync_copy(srcp_hbm.at[wid, 0], sidx[0])
    pltpu.sync_copy(dstp_hbm.at[wid, 0], didx[0])
    pltpu.async_copy(srcp_hbm.at[wid, 1], sidx[1], is_sem[1])
    pltpu.async_copy(dstp_hbm.at[wid, 1], didx[1], id_sem[1])
    plsc.subcore_barrier()

    for s in range(SPB):
        pltpu.async_copy(xs_hbm.at[sidx[0].at[s]], rows[0][s], gsem[0][s])

    def half_block(j, k, fire_next, refill):
        kn = 1 - k
        for s in range(SPB):
            pltpu.make_async_copy(
                xs_hbm.at[sidx[k].at[s]], rows[k][s], gsem[k][s]
            ).wait()
            pltpu.async_copy(rows[k][s], acc.at[didx[k].at[s]], ssem[k][s], add=True)
        if fire_next:
            pltpu.make_async_copy(srcp_hbm.at[wid, 0], sidx[kn], is_sem[kn]).wait()
            pltpu.make_async_copy(dstp_hbm.at[wid, 0], didx[kn], id_sem[kn]).wait()
            for s in range(SPB):
                pltpu.async_copy(
                    xs_hbm.at[sidx[kn].at[s]], rows[kn][s], gsem[kn][s]
                )
        for s in range(SPB):
            pltpu.make_async_copy(rows[k][s], acc.at[didx[k].at[s]], ssem[k][s]).wait()
        if refill:
            jn = j + 2
            pltpu.async_copy(srcp_hbm.at[wid, jn], sidx[k], is_sem[k])
            pltpu.async_copy(dstp_hbm.at[wid, jn], didx[k], id_sem[k])

    def pair(pr, c):
        half_block(2 * pr, 0, fire_next=True, refill=True)
        half_block(2 * pr + 1, 1, fire_next=True, refill=True)
        return c

    lax.fori_loop(0, NP - 1, pair, 0)
    half_block(NBLK - 2, 0, fire_next=True, refill=False)
    half_block(NBLK - 1, 1, fire_next=False, refill=False)
    plsc.subcore_barrier()

    pltpu.sync_copy(
        acc.at[pl.ds(r0, ROWS_PT)], out_hbm.at[cid, pl.ds(r0, ROWS_PT)]
    )

    @pl.when(sid == 0)
    def _():
        t0 = NS * ROWS_PT
        pltpu.sync_copy(
            acc.at[pl.ds(t0, ROW_TAIL)], out_hbm.at[cid, pl.ds(t0, ROW_TAIL)]
        )


def _dinv_body(hists_ref, out_ref):
    deg = jnp.sum(hists_ref[...], axis=0) + 1.0
    out_ref[...] = lax.rsqrt(deg)


_dinv_call = pl.pallas_call(
    _dinv_body,
    out_shape=jax.ShapeDtypeStruct((N_HIST // 128, 128), jnp.float32),
)

BM = 2000


def _mm_body(x_ref, w_ref, dinv_ref, o_ref):
    xw = jnp.dot(x_ref[...], w_ref[...], preferred_element_type=jnp.float32)
    o_ref[...] = xw * dinv_ref[...]


_mm_call = pl.pallas_call(
    _mm_body,
    grid=(N // BM,),
    in_specs=[
        pl.BlockSpec((BM, D), lambda i: (i, 0)),
        pl.BlockSpec((D, D), lambda i: (0, 0)),
        pl.BlockSpec((BM, 1), lambda i: (i, 0)),
    ],
    out_specs=pl.BlockSpec((BM, D), lambda i: (i, 0)),
    out_shape=jax.ShapeDtypeStruct((N, D), jnp.float32),
)


def _post_body(p0_ref, p1_ref, xs_ref, dinv_ref, xin_ref, b_ref, g_ref, beta_ref, o_ref):
    agg = p0_ref[...] + p1_ref[...] - xs_ref[...]
    h = agg * dinv_ref[...] + b_ref[...]
    h = jnp.maximum(h, 0.0) + xin_ref[...]
    mu = jnp.mean(h, axis=-1, keepdims=True)
    d = h - mu
    var = jnp.mean(d * d, axis=-1, keepdims=True)
    o_ref[...] = d * lax.rsqrt(var + 1e-5) * g_ref[...] + beta_ref[...]


_post_call = pl.pallas_call(
    _post_body,
    grid=(N // BM,),
    in_specs=[
        pl.BlockSpec((BM, D), lambda i: (i, 0)),
        pl.BlockSpec((BM, D), lambda i: (i, 0)),
        pl.BlockSpec((BM, D), lambda i: (i, 0)),
        pl.BlockSpec((BM, 1), lambda i: (i, 0)),
        pl.BlockSpec((BM, D), lambda i: (i, 0)),
        pl.BlockSpec((1, D), lambda i: (0, 0)),
        pl.BlockSpec((1, D), lambda i: (0, 0)),
        pl.BlockSpec((1, D), lambda i: (0, 0)),
    ],
    out_specs=pl.BlockSpec((BM, D), lambda i: (i, 0)),
    out_shape=jax.ShapeDtypeStruct((N, D), jnp.float32),
)


@jax.jit
def _run(x, edge_index, Ws, bs, gammas, betas):
    src = edge_index[0]
    dst = edge_index[1]
    pad_src = jnp.zeros((NW, PAD_PW), jnp.int32)
    pad_dst = jnp.broadcast_to(
        N + (jnp.arange(PAD_PW, dtype=jnp.int32) % 8), (NW, PAD_PW)
    )
    srcp = jnp.concatenate([src.reshape(NW, EPW_DEG), pad_src], axis=1).reshape(
        NW, NBLK, SPB, CH
    )
    dstp = jnp.concatenate([dst.reshape(NW, EPW_DEG), pad_dst], axis=1).reshape(
        NW, NBLK, SPB, CH
    )

    deg_kernel, seg_kernel = _sc_kernels()
    hists = deg_kernel(dst)
    dinv2d = _dinv_call(hists.reshape(NW, N_HIST // 128, 128))
    dinv_col = dinv2d.reshape(-1)[:N][:, None]

    for i in range(3):
        xs = _mm_call(x, Ws[i], dinv_col)
        parts = seg_kernel(xs, srcp, dstp)
        x = _post_call(
            parts[0], parts[1], xs, dinv_col, x,
            bs[i][None, :], gammas[i][None, :], betas[i][None, :],
        )
    return x


def kernel(x, edge_index, Ws, bs, gammas, betas):
    return _run(x, edge_index, Ws, bs, gammas, betas)
